# baseline (speedup 1.0000x reference)
import functools
import jax
import jax.numpy as jnp
from jax.experimental import pallas as pl
from jax.experimental.pallas import tpu as pltpu

N = 100000
E = 1600000
H = 32
G = 128
D_IN = 50

BN = 1000
NB = N // BN


def _layer_body(agg_ref, inv_ref, x_ref, wl_ref, b_ref, wr_ref, o_ref, *, act):
    agg = agg_ref[...]
    inv = inv_ref[:, 0:1]
    x = x_ref[...]
    mean = agg * inv
    y = (
        jnp.dot(mean, wl_ref[...], preferred_element_type=jnp.float32)
        + jnp.dot(x, wr_ref[...], preferred_element_type=jnp.float32)
        + b_ref[0:1, :]
    )
    if act == "leaky":
        o_ref[...] = jnp.where(y >= 0, y, 0.01 * y)
    else:
        o_ref[...] = jnp.maximum(y, 0.0)


def _fused_layer(agg, inv8, x, Wl, b, Wr, act):
    fa = agg.shape[1]
    fx = x.shape[1]
    fo = Wl.shape[0]
    wlT = Wl.T
    wrT = Wr.T
    b2 = b.reshape(1, fo)
    return pl.pallas_call(
        functools.partial(_layer_body, act=act),
        grid=(NB,),
        in_specs=[
            pl.BlockSpec((BN, fa), lambda i: (i, 0)),
            pl.BlockSpec((BN, 8), lambda i: (i, 0)),
            pl.BlockSpec((BN, fx), lambda i: (i, 0)),
            pl.BlockSpec((fa, fo), lambda i: (0, 0)),
            pl.BlockSpec((1, fo), lambda i: (0, 0)),
            pl.BlockSpec((fx, fo), lambda i: (0, 0)),
        ],
        out_specs=pl.BlockSpec((BN, fo), lambda i: (i, 0)),
        out_shape=jax.ShapeDtypeStruct((N, fo), jnp.float32),
    )(agg, inv8, x, wlT, b2, wrT)


def _pool_body(x_ref, batch_ref, s_ref, c_ref):
    i = pl.program_id(0)
    x = x_ref[...]
    bvals = batch_ref[:, 0:1]
    gids = jax.lax.broadcasted_iota(jnp.int32, (1, G), 1)
    onehot = (bvals == gids).astype(jnp.float32)
    ps = jnp.dot(onehot.T, x, preferred_element_type=jnp.float32)
    ones = jnp.ones_like(x)
    pc = jnp.dot(onehot.T, ones, preferred_element_type=jnp.float32)

    @pl.when(i == 0)
    def _init():
        s_ref[...] = jnp.zeros_like(s_ref)
        c_ref[...] = jnp.zeros_like(c_ref)

    s_ref[...] += ps
    c_ref[...] += pc


def _pool(x3, batch8):
    return pl.pallas_call(
        _pool_body,
        grid=(NB,),
        in_specs=[
            pl.BlockSpec((BN, 4 * H), lambda i: (i, 0)),
            pl.BlockSpec((BN, 8), lambda i: (i, 0)),
        ],
        out_specs=[
            pl.BlockSpec((G, 4 * H), lambda i: (0, 0)),
            pl.BlockSpec((G, 4 * H), lambda i: (0, 0)),
        ],
        out_shape=[
            jax.ShapeDtypeStruct((G, 4 * H), jnp.float32),
            jax.ShapeDtypeStruct((G, 4 * H), jnp.float32),
        ],
    )(x3, batch8)


def _head_body(s_ref, c_ref, wf1_ref, bf1_ref, wf2_ref, bf2_ref, wo_ref, bo_ref, o_ref):
    xp = s_ref[...] / jnp.maximum(c_ref[...], 1.0)
    x4 = jnp.maximum(
        jnp.dot(xp, wf1_ref[...], preferred_element_type=jnp.float32) + bf1_ref[0:1, :],
        0.0,
    )
    x5 = jnp.maximum(
        jnp.dot(x4, wf2_ref[...], preferred_element_type=jnp.float32) + bf2_ref[0:1, :],
        0.0,
    )
    z = jnp.dot(x5, wo_ref[...], preferred_element_type=jnp.float32) + bo_ref[0:1, :]
    o_ref[...] = jax.nn.sigmoid(z)


def _head(s, c, Wf1, bf1, Wf2, bf2, Wo, bo):
    return pl.pallas_call(
        _head_body,
        out_shape=jax.ShapeDtypeStruct((G, 1), jnp.float32),
    )(
        s,
        c,
        Wf1.T,
        bf1.reshape(1, -1),
        Wf2.T,
        bf2.reshape(1, -1),
        Wo.T,
        bo.reshape(1, 1),
    )


def _aggregate(h, dst_sorted_or_raw, src):
    msg = h[src]
    return jax.ops.segment_sum(msg, dst_sorted_or_raw, num_segments=N)


def kernel(x, edge_index, batch, W1l, b1, W1r, W2l, b2, W2r, W3l, b3, W3r, Wf1, bf1, Wf2, bf2, Wo, bo):
    src = edge_index[0]
    dst = edge_index[1]

    deg = jax.ops.segment_sum(jnp.ones((E,), jnp.float32), dst, num_segments=N)
    inv = 1.0 / jnp.maximum(deg, 1.0)
    inv8 = jnp.broadcast_to(inv[:, None], (N, 8))
    batch8 = jnp.broadcast_to(batch[:, None], (N, 8))

    h1 = x @ W1l.T
    agg1 = _aggregate(h1, dst, src)
    eye32 = jnp.eye(H, dtype=jnp.float32)
    out1 = _fused_layer(agg1, inv8, x, eye32, b1, W1r, "leaky")

    agg2 = _aggregate(out1, dst, src)
    out2 = _fused_layer(agg2, inv8, out1, W2l, b2, W2r, "relu")

    agg3 = _aggregate(out2, dst, src)
    out3 = _fused_layer(agg3, inv8, out2, W3l, b3, W3r, "relu")

    s, c = _pool(out3, batch8)
    return _head(s, c, Wf1, bf1, Wf2, bf2, Wo, bo)


# speedup vs baseline: 2.9214x; 2.9214x over previous
import functools
import jax
import jax.numpy as jnp
from jax.experimental import pallas as pl
from jax.experimental.pallas import tpu as pltpu
from jax.experimental.pallas import tpu_sc as plsc

N = 100000
E = 1600000
H = 32
G = 128
D_IN = 50


BN = 1000
NB = N // BN


def _matmul_body(x_ref, w_ref, o_ref):
    o_ref[...] = jnp.dot(x_ref[...], w_ref[...], preferred_element_type=jnp.float32)


def _matmul(x, wT):
    fi, fo = wT.shape
    return pl.pallas_call(
        _matmul_body,
        grid=(NB,),
        in_specs=[
            pl.BlockSpec((BN, fi), lambda i: (i, 0)),
            pl.BlockSpec((fi, fo), lambda i: (0, 0)),
        ],
        out_specs=pl.BlockSpec((BN, fo), lambda i: (i, 0)),
        out_shape=jax.ShapeDtypeStruct((N, fo), jnp.float32),
    )(x, wT)


def _layer_body(nagg, nx, nout, act, *refs):
    agg_refs = refs[:nagg]
    deg_ref = refs[nagg]
    x_refs = refs[nagg + 1 : nagg + 1 + nx]
    wl_refs = refs[nagg + 1 + nx : nagg + 1 + nx + nagg]
    wr_refs = refs[nagg + 1 + nx + nagg : nagg + 1 + nx + nagg + nx]
    b_ref = refs[nagg + 1 + nx + nagg + nx]
    o_refs = refs[-nout:]

    inv = 1.0 / jnp.maximum(deg_ref[:, 0:1], 1.0)
    y = b_ref[0:1, :]
    for a, w in zip(agg_refs, wl_refs):
        y = y + jnp.dot(a[...] * inv, w[...], preferred_element_type=jnp.float32)
    for x, w in zip(x_refs, wr_refs):
        y = y + jnp.dot(x[...], w[...], preferred_element_type=jnp.float32)
    if act == "leaky":
        y = jnp.where(y >= 0, y, 0.01 * y)
    else:
        y = jnp.maximum(y, 0.0)
    if nout == 1:
        o_refs[0][...] = y
    else:
        half = y.shape[1] // 2
        o_refs[0][...] = y[:, :half]
        o_refs[1][...] = y[:, half:]


def _fused_layer(aggs, deg8, xs, WlTs, WrTs, b, act, nout=1):
    fo = b.shape[0]
    b2 = b.reshape(1, fo)
    in_specs = (
        [pl.BlockSpec((BN, a.shape[1]), lambda i: (i, 0)) for a in aggs]
        + [pl.BlockSpec((BN, 8), lambda i: (i, 0))]
        + [pl.BlockSpec((BN, x.shape[1]), lambda i: (i, 0)) for x in xs]
        + [pl.BlockSpec(w.shape, lambda i: (0, 0)) for w in WlTs]
        + [pl.BlockSpec(w.shape, lambda i: (0, 0)) for w in WrTs]
        + [pl.BlockSpec((1, fo), lambda i: (0, 0))]
    )
    fo_part = fo // nout
    out_specs = [pl.BlockSpec((BN, fo_part), lambda i: (i, 0)) for _ in range(nout)]
    out_shape = [jax.ShapeDtypeStruct((N, fo_part), jnp.float32) for _ in range(nout)]
    if nout == 1:
        out_specs = out_specs[0]
        out_shape = out_shape[0]
    res = pl.pallas_call(
        functools.partial(_layer_body, len(aggs), len(xs), nout, act),
        grid=(NB,),
        in_specs=in_specs,
        out_specs=out_specs,
        out_shape=out_shape,
    )(*aggs, deg8, *xs, *WlTs, *WrTs, b2)
    return res


def _pool_body(x_ref, batch_ref, s_ref, c_ref):
    i = pl.program_id(0)
    x = x_ref[...]
    bvals = batch_ref[:, 0:1]
    gids = jax.lax.broadcasted_iota(jnp.int32, (1, G), 1)
    onehot = (bvals == gids).astype(jnp.float32)
    ps = jnp.dot(onehot.T, x, preferred_element_type=jnp.float32)
    pc = jnp.dot(onehot.T, jnp.ones_like(x), preferred_element_type=jnp.float32)

    @pl.when(i == 0)
    def _init():
        s_ref[...] = jnp.zeros_like(s_ref)
        c_ref[...] = jnp.zeros_like(c_ref)

    s_ref[...] += ps
    c_ref[...] += pc


def _pool(x3, batch8):
    return pl.pallas_call(
        _pool_body,
        grid=(NB,),
        in_specs=[
            pl.BlockSpec((BN, 4 * H), lambda i: (i, 0)),
            pl.BlockSpec((BN, 8), lambda i: (i, 0)),
        ],
        out_specs=[
            pl.BlockSpec((G, 4 * H), lambda i: (0, 0)),
            pl.BlockSpec((G, 4 * H), lambda i: (0, 0)),
        ],
        out_shape=[
            jax.ShapeDtypeStruct((G, 4 * H), jnp.float32),
            jax.ShapeDtypeStruct((G, 4 * H), jnp.float32),
        ],
    )(x3, batch8)


def _head_body(s_ref, c_ref, wf1_ref, bf1_ref, wf2_ref, bf2_ref, wo_ref, bo_ref, o_ref):
    xp = s_ref[...] / jnp.maximum(c_ref[...], 1.0)
    x4 = jnp.maximum(
        jnp.dot(xp, wf1_ref[...], preferred_element_type=jnp.float32) + bf1_ref[0:1, :],
        0.0,
    )
    x5 = jnp.maximum(
        jnp.dot(x4, wf2_ref[...], preferred_element_type=jnp.float32) + bf2_ref[0:1, :],
        0.0,
    )
    z = jnp.dot(x5, wo_ref[...], preferred_element_type=jnp.float32) + bo_ref[0:1, :]
    o_ref[...] = jax.nn.sigmoid(z)


def _head(s, c, Wf1, bf1, Wf2, bf2, Wo, bo):
    return pl.pallas_call(
        _head_body,
        out_shape=jax.ShapeDtypeStruct((G, 1), jnp.float32),
    )(
        s,
        c,
        Wf1.T,
        bf1.reshape(1, -1),
        Wf2.T,
        bf2.reshape(1, -1),
        Wo.T,
        bo.reshape(1, 1),
    )


_C = 128
_TCH = E // _C
_NC = 2
_NS = 16
_NH = N // _NC
_PAD = 16
_ACC = _NH + _PAD
_ZPT = _ACC // _NS
_OPT = 3128
_OPT_LAST = _NH - 15 * _OPT


def _sc_mesh():
    return plsc.VectorSubcoreMesh(core_axis_name="c", subcore_axis_name="s")


def _redirect(dst_v, dloc_v, base):
    for u in range(_C // 16):
        d = dst_v[pl.ds(u * 16, 16)]
        m = (d >= base) & (d < base + _NH)
        dloc_v[pl.ds(u * 16, 16)] = jnp.where(m, d - base, _NH)


def _copy_out(acc, out_hbm, base, sid):
    off = pl.multiple_of(sid * _OPT, 8)

    @pl.when(sid < _NS - 1)
    def _main():
        pltpu.sync_copy(
            acc.at[pl.ds(off, _OPT)],
            out_hbm.at[pl.ds(base + off, _OPT)],
        )

    @pl.when(sid == _NS - 1)
    def _tail():
        pltpu.sync_copy(
            acc.at[pl.ds(15 * _OPT, _OPT_LAST)],
            out_hbm.at[pl.ds(base + 15 * _OPT, _OPT_LAST)],
        )


def _chunk_loop(sid, body):
    nloc = (_TCH - sid + _NS - 1) // _NS

    def b(jj, _):
        body(sid + jj * _NS)
        return 0

    jax.lax.fori_loop(0, nloc, b, 0)


def _sc_segment_sum_make(D):
    @functools.partial(
        pl.kernel,
        mesh=_sc_mesh(),
        compiler_params=pltpu.CompilerParams(use_tc_tiling_on_sc=False),
        out_type=jax.ShapeDtypeStruct((N, D), jnp.float32),
        scratch_types=[
            pltpu.VMEM((_C,), jnp.int32),
            pltpu.VMEM((_C,), jnp.int32),
            pltpu.VMEM((_C,), jnp.int32),
            pltpu.VMEM((_C, D), jnp.float32),
            pltpu.VMEM_SHARED((_ACC, D), jnp.float32),
            pltpu.SemaphoreType.DMA,
        ],
    )
    def k(h_hbm, src_hbm, dst_hbm, zero_hbm, out_hbm, src_v, dst_v, dloc_v, rows_v, acc, sem):
        cid = jax.lax.axis_index("c")
        sid = jax.lax.axis_index("s")
        base = cid * _NH

        pltpu.sync_copy(zero_hbm, acc.at[pl.ds(sid * _ZPT, _ZPT)])
        plsc.subcore_barrier()

        def body(j):
            eoff = pl.multiple_of(j * _C, _C)
            pltpu.sync_copy(src_hbm.at[pl.ds(eoff, _C)], src_v)
            pltpu.sync_copy(dst_hbm.at[pl.ds(eoff, _C)], dst_v)
            _redirect(dst_v, dloc_v, base)
            pltpu.async_copy(h_hbm.at[src_v], rows_v, sem).wait()
            pltpu.sync_copy(rows_v, acc.at[dloc_v], add=True)

        _chunk_loop(sid, body)

        plsc.subcore_barrier()
        _copy_out(acc, out_hbm, base, sid)

    return k


_sc_segment_sum_32 = _sc_segment_sum_make(32)


def _sc_degree_make():
    D = 16

    @functools.partial(
        pl.kernel,
        mesh=_sc_mesh(),
        compiler_params=pltpu.CompilerParams(use_tc_tiling_on_sc=False),
        out_type=jax.ShapeDtypeStruct((N, D), jnp.float32),
        scratch_types=[
            pltpu.VMEM((_C,), jnp.int32),
            pltpu.VMEM((_C,), jnp.int32),
            pltpu.VMEM((_C, D), jnp.float32),
            pltpu.VMEM_SHARED((_ACC, D), jnp.float32),
        ],
    )
    def k(dst_hbm, ones_hbm, zero_hbm, out_hbm, dst_v, dloc_v, ones_v, acc):
        cid = jax.lax.axis_index("c")
        sid = jax.lax.axis_index("s")
        base = cid * _NH

        pltpu.sync_copy(ones_hbm, ones_v)
        pltpu.sync_copy(zero_hbm, acc.at[pl.ds(sid * _ZPT, _ZPT)])
        plsc.subcore_barrier()

        def body(j):
            eoff = pl.multiple_of(j * _C, _C)
            pltpu.sync_copy(dst_hbm.at[pl.ds(eoff, _C)], dst_v)
            _redirect(dst_v, dloc_v, base)
            pltpu.sync_copy(ones_v, acc.at[dloc_v], add=True)

        _chunk_loop(sid, body)

        plsc.subcore_barrier()
        _copy_out(acc, out_hbm, base, sid)

    return k


_sc_degree = _sc_degree_make()


def kernel(x, edge_index, batch, W1l, b1, W1r, W2l, b2, W2r, W3l, b3, W3r, Wf1, bf1, Wf2, bf2, Wo, bo):
    src = edge_index[0]
    dst = edge_index[1]

    ones_c = jnp.ones((_C, 16), jnp.float32)
    zeros32 = jnp.zeros((_ZPT, 32), jnp.float32)
    zeros16 = jnp.zeros((_ZPT, 16), jnp.float32)

    deg16 = _sc_degree(dst, ones_c, zeros16)
    deg8 = deg16[:, :8]
    batch8 = jnp.broadcast_to(batch[:, None], (N, 8))

    h1 = _matmul(x, W1l.T)
    agg1 = _sc_segment_sum_32(h1, src, dst, zeros32)
    eye32 = jnp.eye(H, dtype=jnp.float32)
    out1 = _fused_layer([agg1], deg8, [x], [eye32], [W1r.T], b1, "leaky")

    agg2 = _sc_segment_sum_32(out1, src, dst, zeros32)
    out2a, out2b = _fused_layer(
        [agg2], deg8, [out1], [W2l.T], [W2r.T], b2, "relu", nout=2
    )

    agg3a = _sc_segment_sum_32(out2a, src, dst, zeros32)
    agg3b = _sc_segment_sum_32(out2b, src, dst, zeros32)
    W3lT = W3l.T
    W3rT = W3r.T
    out3 = _fused_layer(
        [agg3a, agg3b],
        deg8,
        [out2a, out2b],
        [W3lT[:H], W3lT[H:]],
        [W3rT[:H], W3rT[H:]],
        b3,
        "relu",
    )

    s, c = _pool(out3, batch8)
    return _head(s, c, Wf1, bf1, Wf2, bf2, Wo, bo)


# speedup vs baseline: 4.0295x; 1.3793x over previous
import functools
import jax
import jax.numpy as jnp
from jax.experimental import pallas as pl
from jax.experimental.pallas import tpu as pltpu
from jax.experimental.pallas import tpu_sc as plsc

N = 100000
E = 1600000
H = 32
G = 128
D_IN = 50


BN = 1000
NB = N // BN


def _matmul_body(x_ref, w_ref, o_ref):
    o_ref[...] = jnp.dot(x_ref[...], w_ref[...], preferred_element_type=jnp.float32)


def _matmul(x, wT):
    fi, fo = wT.shape
    return pl.pallas_call(
        _matmul_body,
        grid=(NB,),
        in_specs=[
            pl.BlockSpec((BN, fi), lambda i: (i, 0)),
            pl.BlockSpec((fi, fo), lambda i: (0, 0)),
        ],
        out_specs=pl.BlockSpec((BN, fo), lambda i: (i, 0)),
        out_shape=jax.ShapeDtypeStruct((N, fo), jnp.float32),
    )(x, wT)


def _layer_body(nagg, nx, nout, act, *refs):
    agg_refs = refs[:nagg]
    deg_ref = refs[nagg]
    x_refs = refs[nagg + 1 : nagg + 1 + nx]
    wl_refs = refs[nagg + 1 + nx : nagg + 1 + nx + nagg]
    wr_refs = refs[nagg + 1 + nx + nagg : nagg + 1 + nx + nagg + nx]
    b_ref = refs[nagg + 1 + nx + nagg + nx]
    o_refs = refs[-nout:]

    inv = 1.0 / jnp.maximum(deg_ref[:, 0:1], 1.0)
    y = b_ref[0:1, :]
    for a, w in zip(agg_refs, wl_refs):
        y = y + jnp.dot(a[...] * inv, w[...], preferred_element_type=jnp.float32)
    for x, w in zip(x_refs, wr_refs):
        y = y + jnp.dot(x[...], w[...], preferred_element_type=jnp.float32)
    if act == "leaky":
        y = jnp.where(y >= 0, y, 0.01 * y)
    else:
        y = jnp.maximum(y, 0.0)
    if nout == 1:
        o_refs[0][...] = y
    else:
        half = y.shape[1] // 2
        o_refs[0][...] = y[:, :half]
        o_refs[1][...] = y[:, half:]


def _fused_layer(aggs, deg8, xs, WlTs, WrTs, b, act, nout=1):
    fo = b.shape[0]
    b2 = b.reshape(1, fo)
    in_specs = (
        [pl.BlockSpec((BN, a.shape[1]), lambda i: (i, 0)) for a in aggs]
        + [pl.BlockSpec((BN, 8), lambda i: (i, 0))]
        + [pl.BlockSpec((BN, x.shape[1]), lambda i: (i, 0)) for x in xs]
        + [pl.BlockSpec(w.shape, lambda i: (0, 0)) for w in WlTs]
        + [pl.BlockSpec(w.shape, lambda i: (0, 0)) for w in WrTs]
        + [pl.BlockSpec((1, fo), lambda i: (0, 0))]
    )
    fo_part = fo // nout
    out_specs = [pl.BlockSpec((BN, fo_part), lambda i: (i, 0)) for _ in range(nout)]
    out_shape = [jax.ShapeDtypeStruct((N, fo_part), jnp.float32) for _ in range(nout)]
    if nout == 1:
        out_specs = out_specs[0]
        out_shape = out_shape[0]
    res = pl.pallas_call(
        functools.partial(_layer_body, len(aggs), len(xs), nout, act),
        grid=(NB,),
        in_specs=in_specs,
        out_specs=out_specs,
        out_shape=out_shape,
    )(*aggs, deg8, *xs, *WlTs, *WrTs, b2)
    return res


def _pool_body(x_ref, batch_ref, s_ref, c_ref):
    i = pl.program_id(0)
    x = x_ref[...]
    bvals = batch_ref[:, 0:1]
    gids = jax.lax.broadcasted_iota(jnp.int32, (1, G), 1)
    onehot = (bvals == gids).astype(jnp.float32)
    ps = jnp.dot(onehot.T, x, preferred_element_type=jnp.float32)
    pc = jnp.dot(onehot.T, jnp.ones_like(x), preferred_element_type=jnp.float32)

    @pl.when(i == 0)
    def _init():
        s_ref[...] = jnp.zeros_like(s_ref)
        c_ref[...] = jnp.zeros_like(c_ref)

    s_ref[...] += ps
    c_ref[...] += pc


def _pool(x3, batch8):
    return pl.pallas_call(
        _pool_body,
        grid=(NB,),
        in_specs=[
            pl.BlockSpec((BN, 4 * H), lambda i: (i, 0)),
            pl.BlockSpec((BN, 8), lambda i: (i, 0)),
        ],
        out_specs=[
            pl.BlockSpec((G, 4 * H), lambda i: (0, 0)),
            pl.BlockSpec((G, 4 * H), lambda i: (0, 0)),
        ],
        out_shape=[
            jax.ShapeDtypeStruct((G, 4 * H), jnp.float32),
            jax.ShapeDtypeStruct((G, 4 * H), jnp.float32),
        ],
    )(x3, batch8)


def _head_body(s_ref, c_ref, wf1_ref, bf1_ref, wf2_ref, bf2_ref, wo_ref, bo_ref, o_ref):
    xp = s_ref[...] / jnp.maximum(c_ref[...], 1.0)
    x4 = jnp.maximum(
        jnp.dot(xp, wf1_ref[...], preferred_element_type=jnp.float32) + bf1_ref[0:1, :],
        0.0,
    )
    x5 = jnp.maximum(
        jnp.dot(x4, wf2_ref[...], preferred_element_type=jnp.float32) + bf2_ref[0:1, :],
        0.0,
    )
    z = jnp.dot(x5, wo_ref[...], preferred_element_type=jnp.float32) + bo_ref[0:1, :]
    o_ref[...] = jax.nn.sigmoid(z)


def _head(s, c, Wf1, bf1, Wf2, bf2, Wo, bo):
    return pl.pallas_call(
        _head_body,
        out_shape=jax.ShapeDtypeStruct((G, 1), jnp.float32),
    )(
        s,
        c,
        Wf1.T,
        bf1.reshape(1, -1),
        Wf2.T,
        bf2.reshape(1, -1),
        Wo.T,
        bo.reshape(1, 1),
    )


_C = 80
_K = 5
_NC = 2
_NS = 16
_EPT = E // _NS
_NBLK = _EPT // (_C * _K)
_NH = N // _NC
_PAD = 16
_ACC = _NH + _PAD
_ZPT = _ACC // _NS
_OPT = 3128
_OPT_LAST = _NH - 15 * _OPT


def _sc_mesh():
    return plsc.VectorSubcoreMesh(core_axis_name="c", subcore_axis_name="s")


def _redirect(dst_v, dloc_v, base):
    for q in range(_K):
        for u in range(_C // 16):
            d = dst_v[pl.ds(q * _C + u * 16, 16)]
            m = (d >= base) & (d < base + _NH)
            dloc_v[q, pl.ds(u * 16, 16)] = jnp.where(m, d - base, _NH)


def _copy_out(acc, out_hbm, base, sid):
    off = pl.multiple_of(sid * _OPT, 8)

    @pl.when(sid < _NS - 1)
    def _main():
        pltpu.sync_copy(
            acc.at[pl.ds(off, _OPT)],
            out_hbm.at[pl.ds(base + off, _OPT)],
        )

    @pl.when(sid == _NS - 1)
    def _tail():
        pltpu.sync_copy(
            acc.at[pl.ds(15 * _OPT, _OPT_LAST)],
            out_hbm.at[pl.ds(base + 15 * _OPT, _OPT_LAST)],
        )


def _sc_segment_sum_make(D):
    @functools.partial(
        pl.kernel,
        mesh=_sc_mesh(),
        compiler_params=pltpu.CompilerParams(use_tc_tiling_on_sc=False),
        out_type=jax.ShapeDtypeStruct((N, D), jnp.float32),
        scratch_types=[
            pltpu.VMEM((_K * _C,), jnp.int32),
            pltpu.VMEM((_K * _C,), jnp.int32),
            pltpu.VMEM((_K, _C), jnp.int32),
            pltpu.VMEM((_K, _C, D), jnp.float32),
            pltpu.VMEM_SHARED((_ACC, D), jnp.float32),
            pltpu.SemaphoreType.DMA,
        ],
    )
    def k(h_hbm, src_hbm, dst_hbm, zero_hbm, out_hbm, src_v, dst_v, dloc_v, rows_v, acc, sem):
        cid = jax.lax.axis_index("c")
        sid = jax.lax.axis_index("s")
        base = cid * _NH
        ebase = sid * _EPT

        pltpu.sync_copy(zero_hbm, acc.at[pl.ds(sid * _ZPT, _ZPT)])
        plsc.subcore_barrier()

        def body(b, _):
            eoff = pl.multiple_of(ebase + b * (_K * _C), _K * _C)
            pltpu.sync_copy(src_hbm.at[pl.ds(eoff, _K * _C)], src_v)
            pltpu.sync_copy(dst_hbm.at[pl.ds(eoff, _K * _C)], dst_v)
            _redirect(dst_v, dloc_v, base)
            copies = [
                pltpu.async_copy(
                    h_hbm.at[src_v.at[pl.ds(q * _C, _C)]], rows_v.at[q], sem
                )
                for q in range(_K)
            ]
            for c in copies:
                c.wait()
            for q in range(_K):
                pltpu.sync_copy(rows_v.at[q], acc.at[dloc_v.at[q]], add=True)
            return 0

        jax.lax.fori_loop(0, _NBLK, body, 0)

        plsc.subcore_barrier()
        _copy_out(acc, out_hbm, base, sid)

    return k


_sc_segment_sum_32 = _sc_segment_sum_make(32)


def _sc_degree_make():
    D = 16

    @functools.partial(
        pl.kernel,
        mesh=_sc_mesh(),
        compiler_params=pltpu.CompilerParams(use_tc_tiling_on_sc=False),
        out_type=jax.ShapeDtypeStruct((N, D), jnp.float32),
        scratch_types=[
            pltpu.VMEM((_K * _C,), jnp.int32),
            pltpu.VMEM((_K, _C), jnp.int32),
            pltpu.VMEM((_C, D), jnp.float32),
            pltpu.VMEM_SHARED((_ACC, D), jnp.float32),
        ],
    )
    def k(dst_hbm, ones_hbm, zero_hbm, out_hbm, dst_v, dloc_v, ones_v, acc):
        cid = jax.lax.axis_index("c")
        sid = jax.lax.axis_index("s")
        base = cid * _NH
        ebase = sid * _EPT

        pltpu.sync_copy(ones_hbm, ones_v)
        pltpu.sync_copy(zero_hbm, acc.at[pl.ds(sid * _ZPT, _ZPT)])
        plsc.subcore_barrier()

        def body(b, _):
            eoff = pl.multiple_of(ebase + b * (_K * _C), _K * _C)
            pltpu.sync_copy(dst_hbm.at[pl.ds(eoff, _K * _C)], dst_v)
            _redirect(dst_v, dloc_v, base)
            for q in range(_K):
                pltpu.sync_copy(ones_v, acc.at[dloc_v.at[q]], add=True)
            return 0

        jax.lax.fori_loop(0, _NBLK, body, 0)

        plsc.subcore_barrier()
        _copy_out(acc, out_hbm, base, sid)

    return k


_sc_degree = _sc_degree_make()


def kernel(x, edge_index, batch, W1l, b1, W1r, W2l, b2, W2r, W3l, b3, W3r, Wf1, bf1, Wf2, bf2, Wo, bo):
    src = edge_index[0]
    dst = edge_index[1]

    ones_c = jnp.ones((_C, 16), jnp.float32)
    zeros32 = jnp.zeros((_ZPT, 32), jnp.float32)
    zeros16 = jnp.zeros((_ZPT, 16), jnp.float32)

    deg16 = _sc_degree(dst, ones_c, zeros16)
    deg8 = deg16[:, :8]
    batch8 = jnp.broadcast_to(batch[:, None], (N, 8))

    h1 = _matmul(x, W1l.T)
    agg1 = _sc_segment_sum_32(h1, src, dst, zeros32)
    eye32 = jnp.eye(H, dtype=jnp.float32)
    out1 = _fused_layer([agg1], deg8, [x], [eye32], [W1r.T], b1, "leaky")

    agg2 = _sc_segment_sum_32(out1, src, dst, zeros32)
    out2a, out2b = _fused_layer(
        [agg2], deg8, [out1], [W2l.T], [W2r.T], b2, "relu", nout=2
    )

    agg3a = _sc_segment_sum_32(out2a, src, dst, zeros32)
    agg3b = _sc_segment_sum_32(out2b, src, dst, zeros32)
    W3lT = W3l.T
    W3rT = W3r.T
    out3 = _fused_layer(
        [agg3a, agg3b],
        deg8,
        [out2a, out2b],
        [W3lT[:H], W3lT[H:]],
        [W3rT[:H], W3rT[H:]],
        b3,
        "relu",
    )

    s, c = _pool(out3, batch8)
    return _head(s, c, Wf1, bf1, Wf2, bf2, Wo, bo)
